# Initial kernel scaffold; baseline (speedup 1.0000x reference)
import functools

import jax
import jax.numpy as jnp
from jax import lax
from jax.experimental import pallas as pl
from jax.experimental.pallas import tpu as pltpu
from jax.experimental.pallas import tpu_sc as plsc

NN = 2048
EE = 65536
DF = 128

NC = 2
NS = 16
NW = NC * NS
LL = 16
ROWS = 32
PASSES = NN // (NW * ROWS)
ECHUNK = 16384

_SC_MESH = plsc.VectorSubcoreMesh(core_axis_name="c", subcore_axis_name="s")


@functools.partial(
    pl.kernel,
    out_type=jax.ShapeDtypeStruct((NN * NN,), jnp.float32),
    mesh=_SC_MESH,
    scratch_types=[
        pltpu.VMEM((ROWS * NN,), jnp.float32),
        pltpu.VMEM((ECHUNK,), jnp.int32),
        pltpu.VMEM((ECHUNK,), jnp.int32),
    ],
)
def _build_mask(src_hbm, dst_hbm, mask_hbm, buf, srcv, dstv):
    cid = lax.axis_index("c")
    sid = lax.axis_index("s")
    wid = sid * NC + cid
    zeros16 = jnp.zeros((LL,), jnp.float32)
    ones16 = jnp.ones((LL,), jnp.float32)
    for p in range(PASSES):
        row_base = (wid * PASSES + p) * ROWS

        def zbody(t, carry):
            buf[pl.ds(t * LL, LL)] = zeros16
            return carry

        lax.fori_loop(0, ROWS * NN // LL, zbody, 0, unroll=8)

        for e0 in range(0, EE, ECHUNK):
            pltpu.sync_copy(src_hbm.at[pl.ds(e0, ECHUNK)], srcv)
            pltpu.sync_copy(dst_hbm.at[pl.ds(e0, ECHUNK)], dstv)

            def ebody(t, carry):
                s16 = srcv[pl.ds(t * LL, LL)]
                d16 = dstv[pl.ds(t * LL, LL)]
                loc = s16 - row_base
                m = (loc >= 0) & (loc < ROWS)
                idx = jnp.where(m, loc * NN + d16, 0)
                plsc.store_scatter(buf, [idx], ones16, mask=m)
                return carry

            lax.fori_loop(0, ECHUNK // LL, ebody, 0, unroll=4)

        out_base = pl.multiple_of(row_base * NN, 8)
        pltpu.sync_copy(buf, mask_hbm.at[pl.ds(out_base, ROWS * NN)])


BI = 512
BJ = 512
GI = NN // BI
GJ = NN // BJ


def _powmask_body(lhs_ref, rhs_ref, mask2_ref, cs1_ref, cs2_ref):
    i = pl.program_id(1)
    lhs = lhs_ref[...]
    rhs = rhs_ref[...]
    c = jax.lax.dot(
        lhs.astype(jnp.bfloat16),
        rhs.astype(jnp.bfloat16),
        preferred_element_type=jnp.float32,
    )
    m2 = (c > 0.0).astype(jnp.float32)
    mask2_ref[...] = m2
    dn = (((0,), (0,)), ((), ()))
    part = jax.lax.dot_general(
        m2, jnp.ones((BI, 1), jnp.float32), dn, preferred_element_type=jnp.float32
    )

    @pl.when(i == 0)
    def _():
        cs1_ref[...] = jax.lax.dot_general(
            rhs, jnp.ones((NN, 1), jnp.float32), dn,
            preferred_element_type=jnp.float32,
        )
        cs2_ref[...] = part

    @pl.when(i != 0)
    def _():
        cs2_ref[...] += part


_powmask = pl.pallas_call(
    _powmask_body,
    grid=(GJ, GI),
    in_specs=[
        pl.BlockSpec((BI, NN), lambda j, i: (i, 0)),
        pl.BlockSpec((NN, BJ), lambda j, i: (0, j)),
    ],
    out_specs=[
        pl.BlockSpec((BI, BJ), lambda j, i: (i, j)),
        pl.BlockSpec((BJ, 1), lambda j, i: (j, 0)),
        pl.BlockSpec((BJ, 1), lambda j, i: (j, 0)),
    ],
    out_shape=[
        jax.ShapeDtypeStruct((NN, NN), jnp.float32),
        jax.ShapeDtypeStruct((NN, 1), jnp.float32),
        jax.ShapeDtypeStruct((NN, 1), jnp.float32),
    ],
)


def _prep_body(x_ref, w_ref, b_ref, alpha_ref, cs1_ref, cs2_ref,
               g1_ref, g2_ref, dinv1_ref, dinv2_ref, bvec_ref):
    h = jnp.dot(x_ref[...], w_ref[...], preferred_element_type=jnp.float32)
    a0 = alpha_ref[0]
    a1 = alpha_ref[1]
    d1 = jax.lax.rsqrt(cs1_ref[...] + 1.0)
    d2 = jax.lax.rsqrt(cs2_ref[...] + 1.0)
    dinv1_ref[...] = d1
    dinv2_ref[...] = d2
    g1_ref[...] = (a0 * d1) * h
    g2_ref[...] = (a1 * d2) * h
    bvec_ref[...] = (a0 + a1) * b_ref[...]


_prep = pl.pallas_call(
    _prep_body,
    in_specs=[
        pl.BlockSpec(memory_space=pltpu.VMEM),
        pl.BlockSpec(memory_space=pltpu.VMEM),
        pl.BlockSpec(memory_space=pltpu.VMEM),
        pl.BlockSpec(memory_space=pltpu.SMEM),
        pl.BlockSpec(memory_space=pltpu.VMEM),
        pl.BlockSpec(memory_space=pltpu.VMEM),
    ],
    out_specs=[
        pl.BlockSpec(memory_space=pltpu.VMEM),
        pl.BlockSpec(memory_space=pltpu.VMEM),
        pl.BlockSpec(memory_space=pltpu.VMEM),
        pl.BlockSpec(memory_space=pltpu.VMEM),
        pl.BlockSpec(memory_space=pltpu.VMEM),
    ],
    out_shape=[
        jax.ShapeDtypeStruct((NN, DF), jnp.float32),
        jax.ShapeDtypeStruct((NN, DF), jnp.float32),
        jax.ShapeDtypeStruct((NN, 1), jnp.float32),
        jax.ShapeDtypeStruct((NN, 1), jnp.float32),
        jax.ShapeDtypeStruct((1, DF), jnp.float32),
    ],
)


def _agg_body(m1_ref, m2_ref, g1_ref, g2_ref, dinv1_ref, dinv2_ref, bvec_ref,
              out_ref):
    j = pl.program_id(0)
    dn = (((0,), (0,)), ((), ()))
    g1 = g1_ref[...]
    g2 = g2_ref[...]
    s1 = jax.lax.dot_general(m1_ref[...], g1, dn, preferred_element_type=jnp.float32)
    s2 = jax.lax.dot_general(m2_ref[...], g2, dn, preferred_element_type=jnp.float32)
    gs1 = g1_ref[pl.ds(j * BJ, BJ), :]
    gs2 = g2_ref[pl.ds(j * BJ, BJ), :]
    out_ref[...] = (dinv1_ref[...] * (s1 + gs1)
                    + dinv2_ref[...] * (s2 + gs2)
                    + bvec_ref[...])


_agg = pl.pallas_call(
    _agg_body,
    grid=(GJ,),
    in_specs=[
        pl.BlockSpec((NN, BJ), lambda j: (0, j)),
        pl.BlockSpec((NN, BJ), lambda j: (0, j)),
        pl.BlockSpec((NN, DF), lambda j: (0, 0)),
        pl.BlockSpec((NN, DF), lambda j: (0, 0)),
        pl.BlockSpec((BJ, 1), lambda j: (j, 0)),
        pl.BlockSpec((BJ, 1), lambda j: (j, 0)),
        pl.BlockSpec((1, DF), lambda j: (0, 0)),
    ],
    out_specs=pl.BlockSpec((BJ, DF), lambda j: (j, 0)),
    out_shape=jax.ShapeDtypeStruct((NN, DF), jnp.float32),
)


def kernel(x, edge_index, W, b, alpha):
    src = edge_index[0]
    dst = edge_index[1]
    mask1 = _build_mask(src, dst).reshape(NN, NN)
    mask2, cs1, cs2 = _powmask(mask1, mask1)
    g1, g2, dinv1, dinv2, bvec = _prep(
        x, W, b.reshape(1, DF), alpha, cs1, cs2
    )
    return _agg(mask1, mask2, g1, g2, dinv1, dinv2, bvec)


# speedup vs baseline: 1015.6600x; 1015.6600x over previous
import functools

import jax
import jax.numpy as jnp
from jax import lax
from jax.experimental import pallas as pl
from jax.experimental.pallas import tpu as pltpu
from jax.experimental.pallas import tpu_sc as plsc

NN = 2048
EE = 65536
DF = 128

NC = 2
NS = 16
NW = NC * NS
LL = 16
ROWS = 32
PASSES = NN // (NW * ROWS)
ECHUNK = 16384

_SC_MESH = plsc.VectorSubcoreMesh(core_axis_name="c", subcore_axis_name="s")


@functools.partial(
    pl.kernel,
    out_type=jax.ShapeDtypeStruct((NN * NN,), jnp.float32),
    mesh=_SC_MESH,
    scratch_types=[
        pltpu.VMEM((ROWS * NN,), jnp.float32),
        pltpu.VMEM((ECHUNK,), jnp.int32),
        pltpu.VMEM((ECHUNK,), jnp.int32),
    ],
    compiler_params=pltpu.CompilerParams(needs_layout_passes=False),
)
def _build_mask(src_hbm, dst_hbm, mask_hbm, buf, srcv, dstv):
    cid = lax.axis_index("c")
    sid = lax.axis_index("s")
    wid = sid * NC + cid
    zeros16 = jnp.zeros((LL,), jnp.float32)
    ones16 = jnp.ones((LL,), jnp.float32)
    for p in range(PASSES):
        row_base = (wid * PASSES + p) * ROWS

        def zbody(t, carry):
            buf[pl.ds(t * LL, LL)] = zeros16
            return carry

        lax.fori_loop(0, ROWS * NN // LL, zbody, 0, unroll=8)

        for e0 in range(0, EE, ECHUNK):
            pltpu.sync_copy(src_hbm.at[pl.ds(e0, ECHUNK)], srcv)
            pltpu.sync_copy(dst_hbm.at[pl.ds(e0, ECHUNK)], dstv)

            def ebody(t, carry):
                s16 = srcv[pl.ds(t * LL, LL)]
                d16 = dstv[pl.ds(t * LL, LL)]
                loc = s16 - row_base
                m = (loc >= 0) & (loc < ROWS)
                idx = jnp.where(m, loc * NN + d16, 0)
                plsc.store_scatter(buf, [idx], ones16, mask=m)
                return carry

            lax.fori_loop(0, ECHUNK // LL, ebody, 0, unroll=4)

        out_base = pl.multiple_of(row_base * NN, 8)
        pltpu.sync_copy(buf, mask_hbm.at[pl.ds(out_base, ROWS * NN)])


BI = 512
BJ = 512
GI = NN // BI
GJ = NN // BJ


def _powmask_body(lhs_ref, rhs_ref, mask2_ref, cs1_ref, cs2_ref):
    i = pl.program_id(1)
    lhs = lhs_ref[...]
    rhs = rhs_ref[...]
    c = jax.lax.dot(
        lhs.astype(jnp.bfloat16),
        rhs.astype(jnp.bfloat16),
        preferred_element_type=jnp.float32,
    )
    m2 = (c > 0.0).astype(jnp.float32)
    mask2_ref[...] = m2
    dn = (((0,), (0,)), ((), ()))
    part = jax.lax.dot_general(
        m2, jnp.ones((BI, 1), jnp.float32), dn, preferred_element_type=jnp.float32
    )

    @pl.when(i == 0)
    def _():
        cs1_ref[...] = jax.lax.dot_general(
            rhs, jnp.ones((NN, 1), jnp.float32), dn,
            preferred_element_type=jnp.float32,
        )
        cs2_ref[...] = part

    @pl.when(i != 0)
    def _():
        cs2_ref[...] += part


_powmask = pl.pallas_call(
    _powmask_body,
    grid=(GJ, GI),
    in_specs=[
        pl.BlockSpec((BI, NN), lambda j, i: (i, 0)),
        pl.BlockSpec((NN, BJ), lambda j, i: (0, j)),
    ],
    out_specs=[
        pl.BlockSpec((BI, BJ), lambda j, i: (i, j)),
        pl.BlockSpec((BJ, 1), lambda j, i: (j, 0)),
        pl.BlockSpec((BJ, 1), lambda j, i: (j, 0)),
    ],
    out_shape=[
        jax.ShapeDtypeStruct((NN, NN), jnp.float32),
        jax.ShapeDtypeStruct((NN, 1), jnp.float32),
        jax.ShapeDtypeStruct((NN, 1), jnp.float32),
    ],
)


def _prep_body(x_ref, w_ref, b_ref, alpha_ref, cs1_ref, cs2_ref,
               g1_ref, g2_ref, dinv1_ref, dinv2_ref, bvec_ref):
    h = jnp.dot(x_ref[...], w_ref[...], preferred_element_type=jnp.float32)
    a0 = alpha_ref[0]
    a1 = alpha_ref[1]
    d1 = jax.lax.rsqrt(cs1_ref[...] + 1.0)
    d2 = jax.lax.rsqrt(cs2_ref[...] + 1.0)
    dinv1_ref[...] = d1
    dinv2_ref[...] = d2
    g1_ref[...] = (a0 * d1) * h
    g2_ref[...] = (a1 * d2) * h
    bvec_ref[...] = (a0 + a1) * b_ref[...]


_prep = pl.pallas_call(
    _prep_body,
    in_specs=[
        pl.BlockSpec(memory_space=pltpu.VMEM),
        pl.BlockSpec(memory_space=pltpu.VMEM),
        pl.BlockSpec(memory_space=pltpu.VMEM),
        pl.BlockSpec(memory_space=pltpu.SMEM),
        pl.BlockSpec(memory_space=pltpu.VMEM),
        pl.BlockSpec(memory_space=pltpu.VMEM),
    ],
    out_specs=[
        pl.BlockSpec(memory_space=pltpu.VMEM),
        pl.BlockSpec(memory_space=pltpu.VMEM),
        pl.BlockSpec(memory_space=pltpu.VMEM),
        pl.BlockSpec(memory_space=pltpu.VMEM),
        pl.BlockSpec(memory_space=pltpu.VMEM),
    ],
    out_shape=[
        jax.ShapeDtypeStruct((NN, DF), jnp.float32),
        jax.ShapeDtypeStruct((NN, DF), jnp.float32),
        jax.ShapeDtypeStruct((NN, 1), jnp.float32),
        jax.ShapeDtypeStruct((NN, 1), jnp.float32),
        jax.ShapeDtypeStruct((1, DF), jnp.float32),
    ],
)


def _agg_body(m1_ref, m2_ref, g1_ref, g2_ref, dinv1_ref, dinv2_ref, bvec_ref,
              out_ref):
    j = pl.program_id(0)
    dn = (((0,), (0,)), ((), ()))
    g1 = g1_ref[...]
    g2 = g2_ref[...]
    s1 = jax.lax.dot_general(m1_ref[...], g1, dn, preferred_element_type=jnp.float32)
    s2 = jax.lax.dot_general(m2_ref[...], g2, dn, preferred_element_type=jnp.float32)
    gs1 = g1_ref[pl.ds(j * BJ, BJ), :]
    gs2 = g2_ref[pl.ds(j * BJ, BJ), :]
    out_ref[...] = (dinv1_ref[...] * (s1 + gs1)
                    + dinv2_ref[...] * (s2 + gs2)
                    + bvec_ref[...])


_agg = pl.pallas_call(
    _agg_body,
    grid=(GJ,),
    in_specs=[
        pl.BlockSpec((NN, BJ), lambda j: (0, j)),
        pl.BlockSpec((NN, BJ), lambda j: (0, j)),
        pl.BlockSpec((NN, DF), lambda j: (0, 0)),
        pl.BlockSpec((NN, DF), lambda j: (0, 0)),
        pl.BlockSpec((BJ, 1), lambda j: (j, 0)),
        pl.BlockSpec((BJ, 1), lambda j: (j, 0)),
        pl.BlockSpec((1, DF), lambda j: (0, 0)),
    ],
    out_specs=pl.BlockSpec((BJ, DF), lambda j: (j, 0)),
    out_shape=jax.ShapeDtypeStruct((NN, DF), jnp.float32),
)


def kernel(x, edge_index, W, b, alpha):
    src = edge_index[0]
    dst = edge_index[1]
    mask1 = _build_mask(src, dst).reshape(NN, NN)
    mask2, cs1, cs2 = _powmask(mask1, mask1)
    g1, g2, dinv1, dinv2, bvec = _prep(
        x, W, b.reshape(1, DF), alpha, cs1, cs2
    )
    return _agg(mask1, mask2, g1, g2, dinv1, dinv2, bvec)


# speedup vs baseline: 1160.3094x; 1.1424x over previous
import functools

import jax
import jax.numpy as jnp
from jax import lax
from jax.experimental import pallas as pl
from jax.experimental.pallas import tpu as pltpu
from jax.experimental.pallas import tpu_sc as plsc

NN = 2048
EE = 65536
DF = 128

NC = 2
NS = 16
NW = NC * NS
LL = 16
ROWS = 32
PASSES = NN // (NW * ROWS)
ECHUNK = 16384

_SC_MESH = plsc.VectorSubcoreMesh(core_axis_name="c", subcore_axis_name="s")

NCH = EE // ECHUNK
FB = EE // NS


@functools.partial(
    pl.kernel,
    out_type=jax.ShapeDtypeStruct((NN * NN,), jnp.float32),
    mesh=_SC_MESH,
    scratch_types=[
        pltpu.VMEM((ROWS * NN,), jnp.float32),
        pltpu.VMEM((ECHUNK,), jnp.int32),
        pltpu.VMEM((ECHUNK,), jnp.int32),
        pltpu.VMEM_SHARED((EE,), jnp.int32),
        pltpu.SemaphoreType.DMA,
        pltpu.SemaphoreType.DMA,
    ],
    compiler_params=pltpu.CompilerParams(needs_layout_passes=False),
)
def _build_mask(src_hbm, dst_hbm, mask_hbm, buf, cha, chb, flat_sh, sem0, sem1):
    cid = lax.axis_index("c")
    sid = lax.axis_index("s")
    wid = sid * NC + cid
    zeros16 = jnp.zeros((LL,), jnp.float32)
    ones16 = jnp.ones((LL,), jnp.float32)

    off = sid * FB
    pltpu.sync_copy(src_hbm.at[pl.ds(off, FB)], cha.at[pl.ds(0, FB)])
    pltpu.sync_copy(dst_hbm.at[pl.ds(off, FB)], chb.at[pl.ds(0, FB)])

    def fbody(t, carry):
        s16 = cha[pl.ds(t * LL, LL)]
        d16 = chb[pl.ds(t * LL, LL)]
        cha[pl.ds(t * LL, LL)] = s16 * NN + d16
        return carry

    lax.fori_loop(0, FB // LL, fbody, 0, unroll=8)
    pltpu.sync_copy(cha.at[pl.ds(0, FB)], flat_sh.at[pl.ds(off, FB)])
    plsc.subcore_barrier()

    for p in range(PASSES):
        row_base = (wid * PASSES + p) * ROWS
        flat_base = row_base * NN

        def zbody(t, carry):
            buf[pl.ds(t * LL, LL)] = zeros16
            return carry

        lax.fori_loop(0, ROWS * NN // LL, zbody, 0, unroll=8)

        pending = pltpu.async_copy(flat_sh.at[pl.ds(0, ECHUNK)], cha, sem0)
        for ci in range(NCH):
            cur = cha if ci % 2 == 0 else chb
            nxt_handle = None
            if ci + 1 < NCH:
                nbuf = chb if ci % 2 == 0 else cha
                nsem = sem1 if ci % 2 == 0 else sem0
                nxt_handle = pltpu.async_copy(
                    flat_sh.at[pl.ds((ci + 1) * ECHUNK, ECHUNK)], nbuf, nsem
                )
            pending.wait()

            def ebody(t, carry):
                f16 = cur[pl.ds(t * LL, LL)]
                loc = f16 - flat_base
                m = (loc >= 0) & (loc < ROWS * NN)
                idx = jnp.where(m, loc, 0)
                plsc.store_scatter(buf, [idx], ones16, mask=m)
                return carry

            lax.fori_loop(0, ECHUNK // LL, ebody, 0, unroll=8)
            pending = nxt_handle

        out_base = pl.multiple_of(flat_base, 8)
        pltpu.sync_copy(buf, mask_hbm.at[pl.ds(out_base, ROWS * NN)])


BI = 512
BJ = 512
GI = NN // BI
GJ = NN // BJ


def _powmask_body(lhs_ref, rhs_ref, mask2_ref, cs1_ref, cs2_ref):
    i = pl.program_id(1)
    lhs = lhs_ref[...]
    rhs = rhs_ref[...]
    c = jax.lax.dot(
        lhs.astype(jnp.bfloat16),
        rhs.astype(jnp.bfloat16),
        preferred_element_type=jnp.float32,
    )
    m2 = (c > 0.0).astype(jnp.float32)
    mask2_ref[...] = m2
    dn = (((0,), (0,)), ((), ()))
    part = jax.lax.dot_general(
        m2, jnp.ones((BI, 1), jnp.float32), dn, preferred_element_type=jnp.float32
    )

    @pl.when(i == 0)
    def _():
        cs1_ref[...] = jax.lax.dot_general(
            rhs, jnp.ones((NN, 1), jnp.float32), dn,
            preferred_element_type=jnp.float32,
        )
        cs2_ref[...] = part

    @pl.when(i != 0)
    def _():
        cs2_ref[...] += part


_powmask = pl.pallas_call(
    _powmask_body,
    grid=(GJ, GI),
    in_specs=[
        pl.BlockSpec((BI, NN), lambda j, i: (i, 0)),
        pl.BlockSpec((NN, BJ), lambda j, i: (0, j)),
    ],
    out_specs=[
        pl.BlockSpec((BI, BJ), lambda j, i: (i, j)),
        pl.BlockSpec((BJ, 1), lambda j, i: (j, 0)),
        pl.BlockSpec((BJ, 1), lambda j, i: (j, 0)),
    ],
    out_shape=[
        jax.ShapeDtypeStruct((NN, NN), jnp.float32),
        jax.ShapeDtypeStruct((NN, 1), jnp.float32),
        jax.ShapeDtypeStruct((NN, 1), jnp.float32),
    ],
)


def _prep_body(x_ref, w_ref, b_ref, alpha_ref, cs1_ref, cs2_ref,
               g1_ref, g2_ref, dinv1_ref, dinv2_ref, bvec_ref):
    h = jnp.dot(x_ref[...], w_ref[...], preferred_element_type=jnp.float32)
    a0 = alpha_ref[0]
    a1 = alpha_ref[1]
    d1 = jax.lax.rsqrt(cs1_ref[...] + 1.0)
    d2 = jax.lax.rsqrt(cs2_ref[...] + 1.0)
    dinv1_ref[...] = d1
    dinv2_ref[...] = d2
    g1_ref[...] = (a0 * d1) * h
    g2_ref[...] = (a1 * d2) * h
    bvec_ref[...] = (a0 + a1) * b_ref[...]


_prep = pl.pallas_call(
    _prep_body,
    in_specs=[
        pl.BlockSpec(memory_space=pltpu.VMEM),
        pl.BlockSpec(memory_space=pltpu.VMEM),
        pl.BlockSpec(memory_space=pltpu.VMEM),
        pl.BlockSpec(memory_space=pltpu.SMEM),
        pl.BlockSpec(memory_space=pltpu.VMEM),
        pl.BlockSpec(memory_space=pltpu.VMEM),
    ],
    out_specs=[
        pl.BlockSpec(memory_space=pltpu.VMEM),
        pl.BlockSpec(memory_space=pltpu.VMEM),
        pl.BlockSpec(memory_space=pltpu.VMEM),
        pl.BlockSpec(memory_space=pltpu.VMEM),
        pl.BlockSpec(memory_space=pltpu.VMEM),
    ],
    out_shape=[
        jax.ShapeDtypeStruct((NN, DF), jnp.float32),
        jax.ShapeDtypeStruct((NN, DF), jnp.float32),
        jax.ShapeDtypeStruct((NN, 1), jnp.float32),
        jax.ShapeDtypeStruct((NN, 1), jnp.float32),
        jax.ShapeDtypeStruct((1, DF), jnp.float32),
    ],
)


def _agg_body(m1_ref, m2_ref, g1_ref, g2_ref, dinv1_ref, dinv2_ref, bvec_ref,
              out_ref):
    j = pl.program_id(0)
    dn = (((0,), (0,)), ((), ()))
    g1 = g1_ref[...]
    g2 = g2_ref[...]
    s1 = jax.lax.dot_general(m1_ref[...], g1, dn, preferred_element_type=jnp.float32)
    s2 = jax.lax.dot_general(m2_ref[...], g2, dn, preferred_element_type=jnp.float32)
    gs1 = g1_ref[pl.ds(j * BJ, BJ), :]
    gs2 = g2_ref[pl.ds(j * BJ, BJ), :]
    out_ref[...] = (dinv1_ref[...] * (s1 + gs1)
                    + dinv2_ref[...] * (s2 + gs2)
                    + bvec_ref[...])


_agg = pl.pallas_call(
    _agg_body,
    grid=(GJ,),
    in_specs=[
        pl.BlockSpec((NN, BJ), lambda j: (0, j)),
        pl.BlockSpec((NN, BJ), lambda j: (0, j)),
        pl.BlockSpec((NN, DF), lambda j: (0, 0)),
        pl.BlockSpec((NN, DF), lambda j: (0, 0)),
        pl.BlockSpec((BJ, 1), lambda j: (j, 0)),
        pl.BlockSpec((BJ, 1), lambda j: (j, 0)),
        pl.BlockSpec((1, DF), lambda j: (0, 0)),
    ],
    out_specs=pl.BlockSpec((BJ, DF), lambda j: (j, 0)),
    out_shape=jax.ShapeDtypeStruct((NN, DF), jnp.float32),
)


def kernel(x, edge_index, W, b, alpha):
    src = edge_index[0]
    dst = edge_index[1]
    mask1 = _build_mask(src, dst).reshape(NN, NN)
    mask2, cs1, cs2 = _powmask(mask1, mask1)
    g1, g2, dinv1, dinv2, bvec = _prep(
        x, W, b.reshape(1, DF), alpha, cs1, cs2
    )
    return _agg(mask1, mask2, g1, g2, dinv1, dinv2, bvec)


# speedup vs baseline: 1256.7047x; 1.0831x over previous
import functools

import jax
import jax.numpy as jnp
from jax import lax
from jax.experimental import pallas as pl
from jax.experimental.pallas import tpu as pltpu
from jax.experimental.pallas import tpu_sc as plsc

NN = 2048
EE = 65536
DF = 128

NC = 2
NS = 16
NW = NC * NS
LL = 16
ROWS = 32
PASSES = NN // (NW * ROWS)
ECHUNK = 16384

_SC_MESH = plsc.VectorSubcoreMesh(core_axis_name="c", subcore_axis_name="s")

EPT = EE // NS
NSTR = EPT // 128
NPASS = 4
QROWS = NN // (NC * NPASS)
QW = QROWS * NN
QPAD = 128
SLICE = QW // NS


@functools.partial(
    pl.kernel,
    out_type=jax.ShapeDtypeStruct((NN * NN,), jnp.float32),
    mesh=_SC_MESH,
    scratch_types=[
        pltpu.VMEM((SLICE,), jnp.float32),
        pltpu.VMEM((EPT,), jnp.int32),
        pltpu.VMEM((EPT,), jnp.int32),
        pltpu.VMEM((NSTR, 128), jnp.int32),
        pltpu.VMEM((128,), jnp.float32),
        pltpu.VMEM_SHARED((QW + QPAD,), jnp.float32),
        pltpu.SemaphoreType.DMA,
    ],
    compiler_params=pltpu.CompilerParams(needs_layout_passes=False),
)
def _build_mask(src_hbm, dst_hbm, mask_hbm, zbuf, srcv, dstv, idx2, ones_v,
                spq, ssem):
    cid = lax.axis_index("c")
    sid = lax.axis_index("s")
    zeros16 = jnp.zeros((LL,), jnp.float32)
    ones16 = jnp.ones((LL,), jnp.float32)

    off = sid * EPT
    pltpu.sync_copy(src_hbm.at[pl.ds(off, EPT)], srcv)
    pltpu.sync_copy(dst_hbm.at[pl.ds(off, EPT)], dstv)
    for t in range(128 // LL):
        ones_v[pl.ds(t * LL, LL)] = ones16

    def zb(t, carry):
        zbuf[pl.ds(t * LL, LL)] = zeros16
        return carry

    lax.fori_loop(0, SLICE // LL, zb, 0, unroll=8)
    sbase = pl.multiple_of(sid * SLICE, 8)
    garb16 = QW + lax.iota(jnp.int32, LL)

    for q in range(NPASS):
        qq = cid * NPASS + q
        base = qq * QW
        pltpu.sync_copy(zbuf, spq.at[pl.ds(sbase, SLICE)])
        plsc.subcore_barrier()

        for r in range(NSTR):

            def ib(t, carry):
                s16 = srcv[pl.ds(r * 128 + t * LL, LL)]
                d16 = dstv[pl.ds(r * 128 + t * LL, LL)]
                rel = (s16 * NN + d16) - base
                m = (rel >= 0) & (rel < QW)
                idx2[r, pl.ds(t * LL, LL)] = jnp.where(m, rel, garb16)
                return carry

            lax.fori_loop(0, 128 // LL, ib, 0, unroll=8)

        handles = [
            pltpu.async_copy(ones_v, spq.at[idx2.at[r]], ssem)
            for r in range(NSTR)
        ]
        for h in handles:
            h.wait()
        plsc.subcore_barrier()

        pltpu.sync_copy(
            spq.at[pl.ds(sbase, SLICE)],
            mask_hbm.at[pl.ds(pl.multiple_of(base + sbase, 8), SLICE)],
        )


BI = 512
BJ = 512
GI = NN // BI
GJ = NN // BJ


def _powmask_body(lhs_ref, rhs_ref, mask2_ref, cs1_ref, cs2_ref):
    i = pl.program_id(1)
    lhs = lhs_ref[...]
    rhs = rhs_ref[...]
    c = jax.lax.dot(
        lhs.astype(jnp.bfloat16),
        rhs.astype(jnp.bfloat16),
        preferred_element_type=jnp.float32,
    )
    m2 = (c > 0.0).astype(jnp.float32)
    mask2_ref[...] = m2
    dn = (((0,), (0,)), ((), ()))
    part = jax.lax.dot_general(
        m2, jnp.ones((BI, 1), jnp.float32), dn, preferred_element_type=jnp.float32
    )

    @pl.when(i == 0)
    def _():
        cs1_ref[...] = jax.lax.dot_general(
            rhs, jnp.ones((NN, 1), jnp.float32), dn,
            preferred_element_type=jnp.float32,
        )
        cs2_ref[...] = part

    @pl.when(i != 0)
    def _():
        cs2_ref[...] += part


_powmask = pl.pallas_call(
    _powmask_body,
    grid=(GJ, GI),
    in_specs=[
        pl.BlockSpec((BI, NN), lambda j, i: (i, 0)),
        pl.BlockSpec((NN, BJ), lambda j, i: (0, j)),
    ],
    out_specs=[
        pl.BlockSpec((BI, BJ), lambda j, i: (i, j)),
        pl.BlockSpec((BJ, 1), lambda j, i: (j, 0)),
        pl.BlockSpec((BJ, 1), lambda j, i: (j, 0)),
    ],
    out_shape=[
        jax.ShapeDtypeStruct((NN, NN), jnp.float32),
        jax.ShapeDtypeStruct((NN, 1), jnp.float32),
        jax.ShapeDtypeStruct((NN, 1), jnp.float32),
    ],
)


def _prep_body(x_ref, w_ref, b_ref, alpha_ref, cs1_ref, cs2_ref,
               g1_ref, g2_ref, dinv1_ref, dinv2_ref, bvec_ref):
    h = jnp.dot(x_ref[...], w_ref[...], preferred_element_type=jnp.float32)
    a0 = alpha_ref[0]
    a1 = alpha_ref[1]
    d1 = jax.lax.rsqrt(cs1_ref[...] + 1.0)
    d2 = jax.lax.rsqrt(cs2_ref[...] + 1.0)
    dinv1_ref[...] = d1
    dinv2_ref[...] = d2
    g1_ref[...] = (a0 * d1) * h
    g2_ref[...] = (a1 * d2) * h
    bvec_ref[...] = (a0 + a1) * b_ref[...]


_prep = pl.pallas_call(
    _prep_body,
    in_specs=[
        pl.BlockSpec(memory_space=pltpu.VMEM),
        pl.BlockSpec(memory_space=pltpu.VMEM),
        pl.BlockSpec(memory_space=pltpu.VMEM),
        pl.BlockSpec(memory_space=pltpu.SMEM),
        pl.BlockSpec(memory_space=pltpu.VMEM),
        pl.BlockSpec(memory_space=pltpu.VMEM),
    ],
    out_specs=[
        pl.BlockSpec(memory_space=pltpu.VMEM),
        pl.BlockSpec(memory_space=pltpu.VMEM),
        pl.BlockSpec(memory_space=pltpu.VMEM),
        pl.BlockSpec(memory_space=pltpu.VMEM),
        pl.BlockSpec(memory_space=pltpu.VMEM),
    ],
    out_shape=[
        jax.ShapeDtypeStruct((NN, DF), jnp.float32),
        jax.ShapeDtypeStruct((NN, DF), jnp.float32),
        jax.ShapeDtypeStruct((NN, 1), jnp.float32),
        jax.ShapeDtypeStruct((NN, 1), jnp.float32),
        jax.ShapeDtypeStruct((1, DF), jnp.float32),
    ],
)


def _agg_body(m1_ref, m2_ref, g1_ref, g2_ref, dinv1_ref, dinv2_ref, bvec_ref,
              out_ref):
    j = pl.program_id(0)
    dn = (((0,), (0,)), ((), ()))
    g1 = g1_ref[...]
    g2 = g2_ref[...]
    s1 = jax.lax.dot_general(m1_ref[...], g1, dn, preferred_element_type=jnp.float32)
    s2 = jax.lax.dot_general(m2_ref[...], g2, dn, preferred_element_type=jnp.float32)
    gs1 = g1_ref[pl.ds(j * BJ, BJ), :]
    gs2 = g2_ref[pl.ds(j * BJ, BJ), :]
    out_ref[...] = (dinv1_ref[...] * (s1 + gs1)
                    + dinv2_ref[...] * (s2 + gs2)
                    + bvec_ref[...])


_agg = pl.pallas_call(
    _agg_body,
    grid=(GJ,),
    in_specs=[
        pl.BlockSpec((NN, BJ), lambda j: (0, j)),
        pl.BlockSpec((NN, BJ), lambda j: (0, j)),
        pl.BlockSpec((NN, DF), lambda j: (0, 0)),
        pl.BlockSpec((NN, DF), lambda j: (0, 0)),
        pl.BlockSpec((BJ, 1), lambda j: (j, 0)),
        pl.BlockSpec((BJ, 1), lambda j: (j, 0)),
        pl.BlockSpec((1, DF), lambda j: (0, 0)),
    ],
    out_specs=pl.BlockSpec((BJ, DF), lambda j: (j, 0)),
    out_shape=jax.ShapeDtypeStruct((NN, DF), jnp.float32),
)


def kernel(x, edge_index, W, b, alpha):
    src = edge_index[0]
    dst = edge_index[1]
    mask1 = _build_mask(src, dst).reshape(NN, NN)
    mask2, cs1, cs2 = _powmask(mask1, mask1)
    g1, g2, dinv1, dinv2, bvec = _prep(
        x, W, b.reshape(1, DF), alpha, cs1, cs2
    )
    return _agg(mask1, mask2, g1, g2, dinv1, dinv2, bvec)


# speedup vs baseline: 1276.0463x; 1.0154x over previous
import functools

import jax
import jax.numpy as jnp
from jax import lax
from jax.experimental import pallas as pl
from jax.experimental.pallas import tpu as pltpu
from jax.experimental.pallas import tpu_sc as plsc

NN = 2048
EE = 65536
DF = 128

NC = 2
NS = 16
NW = NC * NS
LL = 16
ROWS = 32
PASSES = NN // (NW * ROWS)
ECHUNK = 16384

_SC_MESH = plsc.VectorSubcoreMesh(core_axis_name="c", subcore_axis_name="s")

EPT = EE // NS
NSTR = EPT // 128
NPASS = 4
QROWS = NN // (NC * NPASS)
QW = QROWS * NN
QPAD = 128
SLICE = QW // NS


@functools.partial(
    pl.kernel,
    out_type=jax.ShapeDtypeStruct((NN * NN,), jnp.float32),
    mesh=_SC_MESH,
    scratch_types=[
        pltpu.VMEM((SLICE,), jnp.float32),
        pltpu.VMEM((EPT,), jnp.int32),
        pltpu.VMEM((EPT,), jnp.int32),
        pltpu.VMEM((EPT,), jnp.int32),
        pltpu.VMEM((EPT,), jnp.float32),
        pltpu.VMEM_SHARED((QW + QPAD,), jnp.float32),
        pltpu.SemaphoreType.DMA,
    ],
    compiler_params=pltpu.CompilerParams(needs_layout_passes=False),
)
def _build_mask(src_hbm, dst_hbm, mask_hbm, zbuf, srcv, dstv, idx2, ones_v,
                spq, ssem):
    cid = lax.axis_index("c")
    sid = lax.axis_index("s")
    zeros16 = jnp.zeros((LL,), jnp.float32)
    ones16 = jnp.ones((LL,), jnp.float32)

    off = sid * EPT
    pltpu.sync_copy(src_hbm.at[pl.ds(off, EPT)], srcv)
    pltpu.sync_copy(dst_hbm.at[pl.ds(off, EPT)], dstv)
    def ob(t, carry):
        ones_v[pl.ds(t * LL, LL)] = ones16
        return carry

    lax.fori_loop(0, EPT // LL, ob, 0, unroll=8)

    def zb(t, carry):
        zbuf[pl.ds(t * LL, LL)] = zeros16
        return carry

    lax.fori_loop(0, SLICE // LL, zb, 0, unroll=8)
    sbase = pl.multiple_of(sid * SLICE, 8)
    garb16 = QW + lax.iota(jnp.int32, LL)

    for q in range(NPASS):
        qq = cid * NPASS + q
        base = qq * QW
        pltpu.sync_copy(zbuf, spq.at[pl.ds(sbase, SLICE)])
        plsc.subcore_barrier()

        def ib(t, carry):
            s16 = srcv[pl.ds(t * LL, LL)]
            d16 = dstv[pl.ds(t * LL, LL)]
            rel = (s16 * NN + d16) - base
            m = (rel >= 0) & (rel < QW)
            idx2[pl.ds(t * LL, LL)] = jnp.where(m, rel, garb16)
            return carry

        lax.fori_loop(0, EPT // LL, ib, 0, unroll=8)

        pltpu.async_copy(ones_v, spq.at[idx2], ssem).wait()
        plsc.subcore_barrier()

        pltpu.sync_copy(
            spq.at[pl.ds(sbase, SLICE)],
            mask_hbm.at[pl.ds(pl.multiple_of(base + sbase, 8), SLICE)],
        )


BI = 512
BJ = 512
GI = NN // BI
GJ = NN // BJ


def _powmask_body(lhs_ref, rhs_ref, mask2_ref, cs1_ref, cs2_ref):
    i = pl.program_id(1)
    lhs = lhs_ref[...]
    rhs = rhs_ref[...]
    c = jax.lax.dot(
        lhs.astype(jnp.bfloat16),
        rhs.astype(jnp.bfloat16),
        preferred_element_type=jnp.float32,
    )
    m2 = (c > 0.0).astype(jnp.float32)
    mask2_ref[...] = m2
    dn = (((0,), (0,)), ((), ()))
    part = jax.lax.dot_general(
        m2, jnp.ones((BI, 1), jnp.float32), dn, preferred_element_type=jnp.float32
    )

    @pl.when(i == 0)
    def _():
        cs1_ref[...] = jax.lax.dot_general(
            rhs, jnp.ones((NN, 1), jnp.float32), dn,
            preferred_element_type=jnp.float32,
        )
        cs2_ref[...] = part

    @pl.when(i != 0)
    def _():
        cs2_ref[...] += part


_powmask = pl.pallas_call(
    _powmask_body,
    grid=(GJ, GI),
    in_specs=[
        pl.BlockSpec((BI, NN), lambda j, i: (i, 0)),
        pl.BlockSpec((NN, BJ), lambda j, i: (0, j)),
    ],
    out_specs=[
        pl.BlockSpec((BI, BJ), lambda j, i: (i, j)),
        pl.BlockSpec((BJ, 1), lambda j, i: (j, 0)),
        pl.BlockSpec((BJ, 1), lambda j, i: (j, 0)),
    ],
    out_shape=[
        jax.ShapeDtypeStruct((NN, NN), jnp.float32),
        jax.ShapeDtypeStruct((NN, 1), jnp.float32),
        jax.ShapeDtypeStruct((NN, 1), jnp.float32),
    ],
)


def _prep_body(x_ref, w_ref, b_ref, alpha_ref, cs1_ref, cs2_ref,
               g1_ref, g2_ref, dinv1_ref, dinv2_ref, bvec_ref):
    h = jnp.dot(x_ref[...], w_ref[...], preferred_element_type=jnp.float32)
    a0 = alpha_ref[0]
    a1 = alpha_ref[1]
    d1 = jax.lax.rsqrt(cs1_ref[...] + 1.0)
    d2 = jax.lax.rsqrt(cs2_ref[...] + 1.0)
    dinv1_ref[...] = d1
    dinv2_ref[...] = d2
    g1_ref[...] = (a0 * d1) * h
    g2_ref[...] = (a1 * d2) * h
    bvec_ref[...] = (a0 + a1) * b_ref[...]


_prep = pl.pallas_call(
    _prep_body,
    in_specs=[
        pl.BlockSpec(memory_space=pltpu.VMEM),
        pl.BlockSpec(memory_space=pltpu.VMEM),
        pl.BlockSpec(memory_space=pltpu.VMEM),
        pl.BlockSpec(memory_space=pltpu.SMEM),
        pl.BlockSpec(memory_space=pltpu.VMEM),
        pl.BlockSpec(memory_space=pltpu.VMEM),
    ],
    out_specs=[
        pl.BlockSpec(memory_space=pltpu.VMEM),
        pl.BlockSpec(memory_space=pltpu.VMEM),
        pl.BlockSpec(memory_space=pltpu.VMEM),
        pl.BlockSpec(memory_space=pltpu.VMEM),
        pl.BlockSpec(memory_space=pltpu.VMEM),
    ],
    out_shape=[
        jax.ShapeDtypeStruct((NN, DF), jnp.float32),
        jax.ShapeDtypeStruct((NN, DF), jnp.float32),
        jax.ShapeDtypeStruct((NN, 1), jnp.float32),
        jax.ShapeDtypeStruct((NN, 1), jnp.float32),
        jax.ShapeDtypeStruct((1, DF), jnp.float32),
    ],
)


def _agg_body(m1_ref, m2_ref, g1_ref, g2_ref, dinv1_ref, dinv2_ref, bvec_ref,
              out_ref):
    j = pl.program_id(0)
    dn = (((0,), (0,)), ((), ()))
    g1 = g1_ref[...]
    g2 = g2_ref[...]
    s1 = jax.lax.dot_general(m1_ref[...], g1, dn, preferred_element_type=jnp.float32)
    s2 = jax.lax.dot_general(m2_ref[...], g2, dn, preferred_element_type=jnp.float32)
    gs1 = g1_ref[pl.ds(j * BJ, BJ), :]
    gs2 = g2_ref[pl.ds(j * BJ, BJ), :]
    out_ref[...] = (dinv1_ref[...] * (s1 + gs1)
                    + dinv2_ref[...] * (s2 + gs2)
                    + bvec_ref[...])


_agg = pl.pallas_call(
    _agg_body,
    grid=(GJ,),
    in_specs=[
        pl.BlockSpec((NN, BJ), lambda j: (0, j)),
        pl.BlockSpec((NN, BJ), lambda j: (0, j)),
        pl.BlockSpec((NN, DF), lambda j: (0, 0)),
        pl.BlockSpec((NN, DF), lambda j: (0, 0)),
        pl.BlockSpec((BJ, 1), lambda j: (j, 0)),
        pl.BlockSpec((BJ, 1), lambda j: (j, 0)),
        pl.BlockSpec((1, DF), lambda j: (0, 0)),
    ],
    out_specs=pl.BlockSpec((BJ, DF), lambda j: (j, 0)),
    out_shape=jax.ShapeDtypeStruct((NN, DF), jnp.float32),
)


def kernel(x, edge_index, W, b, alpha):
    src = edge_index[0]
    dst = edge_index[1]
    mask1 = _build_mask(src, dst).reshape(NN, NN)
    mask2, cs1, cs2 = _powmask(mask1, mask1)
    g1, g2, dinv1, dinv2, bvec = _prep(
        x, W, b.reshape(1, DF), alpha, cs1, cs2
    )
    return _agg(mask1, mask2, g1, g2, dinv1, dinv2, bvec)


# speedup vs baseline: 1458.3208x; 1.1428x over previous
import functools

import jax
import jax.numpy as jnp
from jax import lax
from jax.experimental import pallas as pl
from jax.experimental.pallas import tpu as pltpu
from jax.experimental.pallas import tpu_sc as plsc

NN = 2048
EE = 65536
DF = 128

NC = 2
NS = 16
NW = NC * NS
LL = 16
ROWS = 32
PASSES = NN // (NW * ROWS)
ECHUNK = 16384

_SC_MESH = plsc.VectorSubcoreMesh(core_axis_name="c", subcore_axis_name="s")

EPT = EE // NS
NSTR = EPT // 128
NPASS = 4
QROWS = NN // (NC * NPASS)
QW = QROWS * NN
QPAD = 128
SLICE = QW // NS


@functools.partial(
    pl.kernel,
    out_type=jax.ShapeDtypeStruct((NN * NN,), jnp.float32),
    mesh=_SC_MESH,
    scratch_types=[
        pltpu.VMEM((SLICE,), jnp.float32),
        pltpu.VMEM((EPT,), jnp.int32),
        pltpu.VMEM((EPT,), jnp.int32),
        pltpu.VMEM((EPT,), jnp.int32),
        pltpu.VMEM((EPT,), jnp.float32),
        pltpu.VMEM_SHARED((QW + QPAD,), jnp.float32),
        pltpu.SemaphoreType.DMA,
    ],
    compiler_params=pltpu.CompilerParams(needs_layout_passes=False),
)
def _build_mask(src_hbm, dst_hbm, mask_hbm, zbuf, srcv, dstv, idx2, ones_v,
                spq, ssem):
    cid = lax.axis_index("c")
    sid = lax.axis_index("s")
    zeros16 = jnp.zeros((LL,), jnp.float32)
    ones16 = jnp.ones((LL,), jnp.float32)

    off = sid * EPT
    pltpu.sync_copy(src_hbm.at[pl.ds(off, EPT)], srcv)
    pltpu.sync_copy(dst_hbm.at[pl.ds(off, EPT)], dstv)
    def ob(t, carry):
        ones_v[pl.ds(t * LL, LL)] = ones16
        return carry

    lax.fori_loop(0, EPT // LL, ob, 0, unroll=8)

    def zb(t, carry):
        zbuf[pl.ds(t * LL, LL)] = zeros16
        return carry

    lax.fori_loop(0, SLICE // LL, zb, 0, unroll=8)
    sbase = pl.multiple_of(sid * SLICE, 8)
    garb16 = QW + lax.iota(jnp.int32, LL)

    for q in range(NPASS):
        qq = cid * NPASS + q
        base = qq * QW
        pltpu.sync_copy(zbuf, spq.at[pl.ds(sbase, SLICE)])
        plsc.subcore_barrier()

        def ib(t, carry):
            s16 = srcv[pl.ds(t * LL, LL)]
            d16 = dstv[pl.ds(t * LL, LL)]
            rel = (s16 * NN + d16) - base
            m = (rel >= 0) & (rel < QW)
            idx2[pl.ds(t * LL, LL)] = jnp.where(m, rel, garb16)
            return carry

        lax.fori_loop(0, EPT // LL, ib, 0, unroll=8)

        pltpu.async_copy(ones_v, spq.at[idx2], ssem).wait()
        plsc.subcore_barrier()

        pltpu.sync_copy(
            spq.at[pl.ds(sbase, SLICE)],
            mask_hbm.at[pl.ds(pl.multiple_of(base + sbase, 8), SLICE)],
        )


BI2 = 256
GI2 = NN // BI2
BJ = 512
GJ = NN // BJ
_DN0 = (((0,), (0,)), ((), ()))


def _powmask_body(m1_ref, mask2_ref, cs1_ref, cs2_ref, mbf_ref):
    i = pl.program_id(0)

    @pl.when(i == 0)
    def _():
        mbf_ref[...] = m1_ref[...].astype(jnp.bfloat16)
        cs1_ref[...] = jax.lax.dot_general(
            m1_ref[...], jnp.ones((NN, 1), jnp.float32), _DN0,
            preferred_element_type=jnp.float32,
        )

    lhs = mbf_ref[pl.ds(i * BI2, BI2), :]
    c = jax.lax.dot(lhs, mbf_ref[...], preferred_element_type=jnp.float32)
    m2f = (c > 0.0).astype(jnp.float32)
    mask2_ref[...] = m2f.astype(jnp.bfloat16)
    part = jax.lax.dot_general(
        m2f, jnp.ones((BI2, 1), jnp.float32), _DN0,
        preferred_element_type=jnp.float32,
    )

    @pl.when(i == 0)
    def _():
        cs2_ref[...] = part

    @pl.when(i != 0)
    def _():
        cs2_ref[...] += part


_powmask = pl.pallas_call(
    _powmask_body,
    grid=(GI2,),
    in_specs=[
        pl.BlockSpec((NN, NN), lambda i: (0, 0)),
    ],
    out_specs=[
        pl.BlockSpec((BI2, NN), lambda i: (i, 0)),
        pl.BlockSpec((NN, 1), lambda i: (0, 0)),
        pl.BlockSpec((NN, 1), lambda i: (0, 0)),
    ],
    out_shape=[
        jax.ShapeDtypeStruct((NN, NN), jnp.bfloat16),
        jax.ShapeDtypeStruct((NN, 1), jnp.float32),
        jax.ShapeDtypeStruct((NN, 1), jnp.float32),
    ],
    scratch_shapes=[pltpu.VMEM((NN, NN), jnp.bfloat16)],
)


def _agg_body(m1_ref, m2_ref, x_ref, w_ref, b_ref, alpha_ref, cs1_ref,
              cs2_ref, out_ref):
    j = pl.program_id(0)
    a0 = alpha_ref[0]
    a1 = alpha_ref[1]
    h = jnp.dot(x_ref[...], w_ref[...], preferred_element_type=jnp.float32)
    d1 = jax.lax.rsqrt(cs1_ref[...] + 1.0)
    d2 = jax.lax.rsqrt(cs2_ref[...] + 1.0)
    g1 = (a0 * d1) * h
    g2 = (a1 * d2) * h
    s1 = jax.lax.dot_general(m1_ref[...], g1, _DN0,
                             preferred_element_type=jnp.float32)
    s2 = jax.lax.dot_general(m2_ref[...].astype(jnp.float32), g2, _DN0,
                             preferred_element_type=jnp.float32)
    d1j = jax.lax.rsqrt(cs1_ref[pl.ds(j * BJ, BJ), :] + 1.0)
    d2j = jax.lax.rsqrt(cs2_ref[pl.ds(j * BJ, BJ), :] + 1.0)
    hj = jnp.dot(x_ref[pl.ds(j * BJ, BJ), :], w_ref[...],
                 preferred_element_type=jnp.float32)
    out_ref[...] = (d1j * (s1 + (a0 * d1j) * hj)
                    + d2j * (s2 + (a1 * d2j) * hj)
                    + (a0 + a1) * b_ref[...])


_agg = pl.pallas_call(
    _agg_body,
    grid=(GJ,),
    in_specs=[
        pl.BlockSpec((NN, BJ), lambda j: (0, j)),
        pl.BlockSpec((NN, BJ), lambda j: (0, j)),
        pl.BlockSpec((NN, DF), lambda j: (0, 0)),
        pl.BlockSpec((DF, DF), lambda j: (0, 0)),
        pl.BlockSpec((1, DF), lambda j: (0, 0)),
        pl.BlockSpec(memory_space=pltpu.SMEM),
        pl.BlockSpec((NN, 1), lambda j: (0, 0)),
        pl.BlockSpec((NN, 1), lambda j: (0, 0)),
    ],
    out_specs=pl.BlockSpec((BJ, DF), lambda j: (j, 0)),
    out_shape=jax.ShapeDtypeStruct((NN, DF), jnp.float32),
)


def kernel(x, edge_index, W, b, alpha):
    src = edge_index[0]
    dst = edge_index[1]
    mask1 = _build_mask(src, dst).reshape(NN, NN)
    mask2, cs1, cs2 = _powmask(mask1)
    return _agg(mask1, mask2, x, W, b.reshape(1, DF), alpha, cs1, cs2)
